# baseline (speedup 1.0000x reference)
import jax
import jax.numpy as jnp
from jax import lax
from jax.experimental import pallas as pl
from jax.experimental.pallas import tpu as pltpu
from jax.experimental.pallas import tpu_sc as plsc

N = 10000
NPAD = 10240
E = 320000
G = 64
C_OUT = 128
NT = 16
PT = NPAD // NT
ET = E // NT
VPT = PT // 16
WCH = 160
K1 = 0x9E3779B9


def _mixv(u):
    u = (u ^ (u >> 16)) * jnp.uint32(0x7FEB352D)
    u = (u ^ (u >> 15)) * jnp.uint32(0x846CA68B)
    return u ^ (u >> 16)


def _u32(x):
    return plsc.bitcast(x, jnp.uint32)


def _i32(x):
    return plsc.bitcast(x, jnp.int32)


_DNUMS = lax.GatherDimensionNumbers(
    offset_dims=(), collapsed_slice_dims=(0,), start_index_map=(0,))


def _take(v, idx):
    return lax.gather(v, idx[:, None], dimension_numbers=_DNUMS,
                      slice_sizes=(1,),
                      mode=lax.GatherScatterMode.PROMISE_IN_BOUNDS)


def _body(x_hbm, src_hbm, dst_hbm, batch_hbm,
          tbl_hbm,
          zsrc_v, hfull_v, aggp_v, srcv, dstv,
          slice_v, key_v, val_v, dsti_v, col_v, batch_v,
          grid_v, hist_v, offs_v, tmp16_v, zsem, psem,
          part_hbm,
          colors_s, h_s, keyA_s, valA_s, keyB_s, valB_s,
          ghist_s, tcount_s, table_s):
    t = lax.axis_index("s")
    lane = lax.iota(jnp.int32, 16)
    ones = jnp.ones((16,), jnp.int32)
    zeros = jnp.zeros((16,), jnp.int32)
    lanem1 = jnp.maximum(lane - 1, 0)
    lanep1 = jnp.minimum(lane + 1, 15)
    l15 = jnp.full((16,), 15, jnp.int32)

    def loop(n, body):
        lax.fori_loop(0, n, lambda i, c: (body(i), 0)[1], 0)

    def runs(sortkey):
        srt = lax.sort(sortkey)
        dig = srt >> 4
        sl = srt & 15
        prev = _take(dig, lanem1)
        isstart = jnp.logical_or(lane == 0, dig != prev)
        st = plsc.cummax(jnp.where(isstart, lane, zeros))
        nxt = _take(dig, lanep1)
        isend = jnp.logical_or(lane == 15, dig != nxt)
        return dig, sl, lane - st, isend

    LAST = N - 15 * PT

    @pl.when(t < 15)
    def _cf():
        pltpu.sync_copy(x_hbm.at[pl.ds(t * PT, PT)], slice_v)
        pltpu.sync_copy(batch_hbm.at[pl.ds(t * PT, PT)], batch_v)

    @pl.when(t == 15)
    def _cl():
        pltpu.sync_copy(x_hbm.at[pl.ds(15 * PT, LAST)],
                        slice_v.at[pl.ds(0, LAST)])
        pltpu.sync_copy(batch_hbm.at[pl.ds(15 * PT, LAST)],
                        batch_v.at[pl.ds(0, LAST)])
    pltpu.sync_copy(slice_v, colors_s.at[pl.ds(t * PT, PT)])
    pltpu.sync_copy(src_hbm.at[pl.ds(t * ET, ET)], srcv)
    pltpu.sync_copy(dst_hbm.at[pl.ds(t * ET, ET)], dstv)
    loop(NPAD // 16, lambda k: zsrc_v.__setitem__(pl.ds(k * 16, 16), zeros))
    zcps = [pltpu.async_copy(
        zsrc_v, table_s.at[pl.ds(t * 4 * NPAD + j * NPAD, NPAD)], zsem)
        for j in range(4)]

    @pl.when(t == 0)
    def _zt0():
        pltpu.sync_copy(zsrc_v.at[pl.ds(0, 8)], table_s.at[pl.ds(G * NPAD, 8)])
    plsc.subcore_barrier()

    def layer(_l, carry):
        pltpu.sync_copy(colors_s.at[pl.ds(t * PT, PT)], slice_v)

        def l1(k):
            v = slice_v[pl.ds(k * 16, 16)]
            key_v[pl.ds(k * 16, 16)] = _i32(_mixv(_u32(v)))
        loop(VPT, l1)
        pltpu.sync_copy(key_v, h_s.at[pl.ds(t * PT, PT)])
        plsc.subcore_barrier()
        pltpu.sync_copy(h_s, hfull_v)

        loop(NPAD // 16, lambda k: aggp_v.__setitem__(pl.ds(k * 16, 16), zeros))

        def l2(i):
            s = srcv[pl.ds(i * 16, 16)]
            d = dstv[pl.ds(i * 16, 16)]
            nb = plsc.load_gather(hfull_v, [s])
            plsc.addupdate_scatter(aggp_v, [d], nb)
        lax.fori_loop(0, ET // 16, lambda i, c: (l2(i), 0)[1], 0, unroll=8)

        pltpu.sync_copy(aggp_v, part_hbm.at[pl.ds(t * NPAD, NPAD)])
        plsc.subcore_barrier()
        cps = [pltpu.async_copy(
            part_hbm.at[pl.ds(p * NPAD + t * PT, PT)],
            aggp_v.at[pl.ds(p * PT, PT)], psem) for p in range(NT)]
        for cp in cps:
            cp.wait()

        def l3(k):
            acc = aggp_v[pl.ds(k * 16, 16)]
            for p in range(1, NT):
                acc = acc + aggp_v[pl.ds(p * PT + k * 16, 16)]
            dsti_v[pl.ds(k * 16, 16)] = acc
        lax.fori_loop(0, VPT, lambda i, c: (l3(i), 0)[1], 0, unroll=4)


        def l4(k):
            a = dsti_v[pl.ds(k * 16, 16)]
            h = hfull_v[pl.ds(t * PT + k * 16, 16)]
            sig = _u32(h) * jnp.uint32(K1) + _mixv(_u32(a))
            gid = t * PT + k * 16 + lane
            sig = jnp.where(gid < N, sig, jnp.uint32(0xFFFFFFFF))
            key_v[pl.ds(k * 16, 16)] = _i32(sig)
            val_v[pl.ds(k * 16, 16)] = gid
        loop(VPT, l4)
        pltpu.sync_copy(key_v, keyA_s.at[pl.ds(t * PT, PT)])
        pltpu.sync_copy(val_v, valA_s.at[pl.ds(t * PT, PT)])

        for p in range(4):
            inK = keyA_s if p % 2 == 0 else keyB_s
            inV = valA_s if p % 2 == 0 else valB_s
            outK = keyB_s if p % 2 == 0 else keyA_s
            outV = valB_s if p % 2 == 0 else valA_s
            sh = 8 * p
            pltpu.sync_copy(inK.at[pl.ds(t * PT, PT)], key_v)
            pltpu.sync_copy(inV.at[pl.ds(t * PT, PT)], val_v)
            hist_v[pl.ds(0, 16)] = zeros
            loop(256 // 16,
                 lambda k: hist_v.__setitem__(pl.ds(k * 16, 16), zeros))

            def s1(k, _sh=sh):
                kv = key_v[pl.ds(k * 16, 16)]
                d = _i32((_u32(kv) >> _sh) & jnp.uint32(255))
                dig, _, rk, isend = runs(d * 16 + lane)
                plsc.addupdate_scatter(hist_v, [dig], rk + 1, mask=isend)
            lax.fori_loop(0, VPT, lambda i, c: (s1(i), 0)[1], 0, unroll=4)
            pltpu.sync_copy(hist_v, ghist_s.at[pl.ds(t * 256, 256)])
            plsc.subcore_barrier()
            pltpu.sync_copy(ghist_s, grid_v)
            rcar = zeros
            for c in range(16):
                def s2(tp, a, _c=c):
                    accp, acct = a
                    v = grid_v[pl.ds(tp * 256 + _c * 16, 16)]
                    return (accp + jnp.where(tp < t, v, zeros), acct + v)
                accp, acct = lax.fori_loop(0, NT, s2, (zeros, zeros),
                                           unroll=4)
                cs = plsc.cumsum(acct)
                offs_v[pl.ds(c * 16, 16)] = accp + cs - acct + rcar
                rcar = rcar + _take(cs, l15)

            def s3(k, _sh=sh):
                kv = key_v[pl.ds(k * 16, 16)]
                d = _i32((_u32(kv) >> _sh) & jnp.uint32(255))
                dig, sl, rk, isend = runs(d * 16 + lane)
                base = plsc.load_gather(offs_v, [dig])
                plsc.addupdate_scatter(offs_v, [dig], rk + 1, mask=isend)
                plsc.store_scatter(dsti_v, [k * 16 + sl], base + rk)
            lax.fori_loop(0, VPT, lambda i, c: (s3(i), 0)[1], 0, unroll=4)
            pltpu.sync_copy(key_v, outK.at[dsti_v])
            pltpu.sync_copy(val_v, outV.at[dsti_v])
            plsc.subcore_barrier()

        pltpu.sync_copy(keyA_s.at[pl.ds(t * PT, PT)], key_v)
        pltpu.sync_copy(valA_s.at[pl.ds(t * PT, PT)], val_v)
        off0 = pl.multiple_of(jnp.maximum(t * PT - 16, 0), 16)
        pltpu.sync_copy(keyA_s.at[pl.ds(off0, 16)], tmp16_v)
        kv0 = key_v[pl.ds(0, 16)]
        tl = tmp16_v[...]
        prev0 = jnp.where(t > 0, tl[15], kv0[0])

        def l6(k, c):
            runs_, prevs = c
            kv = key_v[pl.ds(k * 16, 16)]
            prevv = jnp.where(lane == 0, prevs, _take(kv, lanem1))
            nf = jnp.where(kv != prevv, ones, zeros)
            runv = runs_ + plsc.cumsum(nf)
            dsti_v[pl.ds(k * 16, 16)] = runv
            return _take(runv, l15), _take(kv, l15)
        cnt, _ = lax.fori_loop(0, VPT, l6,
                               (zeros, zeros + prev0))
        tmp16_v[...] = cnt
        pltpu.sync_copy(tmp16_v.at[pl.ds(0, 8)], tcount_s.at[pl.ds(t * 8, 8)])
        plsc.subcore_barrier()
        pltpu.sync_copy(tcount_s, hist_v.at[pl.ds(0, 128)])
        tcv = plsc.load_gather(hist_v, [lane * 8])
        off = jnp.sum(jnp.where(lane < t, tcv, zeros))
        loop(VPT, lambda k: dsti_v.__setitem__(
            pl.ds(k * 16, 16), dsti_v[pl.ds(k * 16, 16)] + off))
        pltpu.sync_copy(dsti_v, colors_s.at[val_v])
        plsc.subcore_barrier()
        return carry

    lax.fori_loop(0, 3, layer, 0)

    for zc in zcps:
        zc.wait()
    plsc.subcore_barrier()

    pltpu.sync_copy(colors_s.at[pl.ds(t * PT, PT)], col_v)
    loop(VPT, lambda k: key_v.__setitem__(pl.ds(k * 16, 16), ones))

    def f2(k):
        c = col_v[pl.ds(k * 16, 16)]
        b = batch_v[pl.ds(k * 16, 16)]
        gid = t * PT + k * 16 + lane
        idx = jnp.where(gid < N, b * NPAD + c, G * NPAD)
        dsti_v[pl.ds(k * 16, 16)] = idx
    loop(VPT, f2)
    pltpu.sync_copy(key_v, table_s.at[dsti_v], add=True)
    plsc.subcore_barrier()

    pltpu.sync_copy(table_s.at[pl.ds(t * 4 * NPAD, 4 * NPAD)],
                    tbl_hbm.at[pl.ds(t * 4 * NPAD, 4 * NPAD)])


@jax.jit
def _run(x, src, dst, batch):
    mesh = plsc.VectorSubcoreMesh(
        core_axis_name="c", subcore_axis_name="s", num_cores=1)
    f = pl.kernel(
        _body,
        out_type=jax.ShapeDtypeStruct((G * NPAD,), jnp.int32),
        mesh=mesh,
        compiler_params=pltpu.CompilerParams(needs_layout_passes=False),
        scratch_types=[
            pltpu.VMEM((NPAD,), jnp.int32),
            pltpu.VMEM((NPAD,), jnp.int32),
            pltpu.VMEM((NPAD,), jnp.int32),
            pltpu.VMEM((ET,), jnp.int32),
            pltpu.VMEM((ET,), jnp.int32),
            pltpu.VMEM((PT,), jnp.int32),
            pltpu.VMEM((PT,), jnp.int32),
            pltpu.VMEM((PT,), jnp.int32),
            pltpu.VMEM((PT,), jnp.int32),
            pltpu.VMEM((PT,), jnp.int32),
            pltpu.VMEM((PT,), jnp.int32),
            pltpu.VMEM((4096,), jnp.int32),
            pltpu.VMEM((256,), jnp.int32),
            pltpu.VMEM((256,), jnp.int32),
            pltpu.VMEM((16,), jnp.int32),
            pltpu.SemaphoreType.DMA,
            pltpu.SemaphoreType.DMA,
            pltpu.HBM((NT * NPAD,), jnp.int32),
            pltpu.VMEM_SHARED((NPAD,), jnp.int32),
            pltpu.VMEM_SHARED((NPAD,), jnp.int32),
            pltpu.VMEM_SHARED((NPAD,), jnp.int32),
            pltpu.VMEM_SHARED((NPAD,), jnp.int32),
            pltpu.VMEM_SHARED((NPAD,), jnp.int32),
            pltpu.VMEM_SHARED((NPAD,), jnp.int32),
            pltpu.VMEM_SHARED((4096,), jnp.int32),
            pltpu.VMEM_SHARED((128,), jnp.int32),
            pltpu.VMEM_SHARED((G * NPAD + 8,), jnp.int32),
        ],
    )
    return f(x, src, dst, batch)


def _tc_body(hist_ref, w_ref, o_ref):
    h = hist_ref[...].astype(jnp.float32)
    nrm2 = jnp.sum(h * h, axis=1, keepdims=True)
    out = jnp.dot(h[:, :N], w_ref[...], preferred_element_type=jnp.float32,
                  precision=lax.Precision.HIGHEST)
    nrm = jnp.sqrt(nrm2)
    o_ref[...] = out / jnp.where(nrm > 0.0, nrm, 1.0)


@jax.jit
def _project(hist, w_pad):
    return pl.pallas_call(
        _tc_body,
        out_shape=jax.ShapeDtypeStruct((G, C_OUT), jnp.float32),
    )(hist, w_pad)


def kernel(x, edge_index, batch, W, b):
    src = edge_index[0].astype(jnp.int32)
    dst = edge_index[1].astype(jnp.int32)
    tbl = _run(x.astype(jnp.int32), src, dst, batch.astype(jnp.int32))
    return _project(tbl.reshape(G, NPAD), W) + b


# speedup vs baseline: 1.0081x; 1.0081x over previous
import jax
import jax.numpy as jnp
from jax import lax
from jax.experimental import pallas as pl
from jax.experimental.pallas import tpu as pltpu
from jax.experimental.pallas import tpu_sc as plsc

N = 10000
NPAD = 10240
E = 320000
G = 64
C_OUT = 128
NT = 16
PT = NPAD // NT
ET = E // NT
VPT = PT // 16
WCH = 160
K1 = 0x9E3779B9


def _mixv(u):
    u = (u ^ (u >> 16)) * jnp.uint32(0x7FEB352D)
    u = (u ^ (u >> 15)) * jnp.uint32(0x846CA68B)
    return u ^ (u >> 16)


def _u32(x):
    return plsc.bitcast(x, jnp.uint32)


def _i32(x):
    return plsc.bitcast(x, jnp.int32)


_DNUMS = lax.GatherDimensionNumbers(
    offset_dims=(), collapsed_slice_dims=(0,), start_index_map=(0,))


def _take(v, idx):
    return lax.gather(v, idx[:, None], dimension_numbers=_DNUMS,
                      slice_sizes=(1,),
                      mode=lax.GatherScatterMode.PROMISE_IN_BOUNDS)


def _body(x_hbm, src_hbm, dst_hbm, batch_hbm,
          tbl_hbm,
          zsrc_v, hfull_v, aggp_v, srcv, dstv,
          slice_v, key_v, val_v, dsti_v, col_v, batch_v,
          grid_v, hist_v, offs_v, tmp16_v, zsem, psem,
          part_hbm,
          colors_s, h_s, keyA_s, valA_s, keyB_s, valB_s,
          ghist_s, tcount_s, table_s):
    t = lax.axis_index("s")
    lane = lax.iota(jnp.int32, 16)
    ones = jnp.ones((16,), jnp.int32)
    zeros = jnp.zeros((16,), jnp.int32)
    lanem1 = jnp.maximum(lane - 1, 0)
    lanep1 = jnp.minimum(lane + 1, 15)
    l15 = jnp.full((16,), 15, jnp.int32)

    def loop(n, body):
        lax.fori_loop(0, n, lambda i, c: (body(i), 0)[1], 0)

    def runs(sortkey):
        srt = lax.sort(sortkey)
        dig = srt >> 4
        sl = srt & 15
        prev = _take(dig, lanem1)
        isstart = jnp.logical_or(lane == 0, dig != prev)
        st = plsc.cummax(jnp.where(isstart, lane, zeros))
        nxt = _take(dig, lanep1)
        isend = jnp.logical_or(lane == 15, dig != nxt)
        return dig, sl, lane - st, isend

    LAST = N - 15 * PT

    @pl.when(t < 15)
    def _cf():
        pltpu.sync_copy(x_hbm.at[pl.ds(t * PT, PT)], slice_v)
        pltpu.sync_copy(batch_hbm.at[pl.ds(t * PT, PT)], batch_v)

    @pl.when(t == 15)
    def _cl():
        pltpu.sync_copy(x_hbm.at[pl.ds(15 * PT, LAST)],
                        slice_v.at[pl.ds(0, LAST)])
        pltpu.sync_copy(batch_hbm.at[pl.ds(15 * PT, LAST)],
                        batch_v.at[pl.ds(0, LAST)])
    pltpu.sync_copy(slice_v, colors_s.at[pl.ds(t * PT, PT)])
    pltpu.sync_copy(src_hbm.at[pl.ds(t * ET, ET)], srcv)
    pltpu.sync_copy(dst_hbm.at[pl.ds(t * ET, ET)], dstv)
    loop(NPAD // 16, lambda k: zsrc_v.__setitem__(pl.ds(k * 16, 16), zeros))
    zcps = [pltpu.async_copy(
        zsrc_v, table_s.at[pl.ds(t * 4 * NPAD + j * NPAD, NPAD)], zsem)
        for j in range(4)]

    @pl.when(t == 0)
    def _zt0():
        pltpu.sync_copy(zsrc_v.at[pl.ds(0, 8)], table_s.at[pl.ds(G * NPAD, 8)])
    plsc.subcore_barrier()

    def layer(_l, carry):
        pltpu.sync_copy(colors_s.at[pl.ds(t * PT, PT)], slice_v)

        def l1(k):
            v = slice_v[pl.ds(k * 16, 16)]
            key_v[pl.ds(k * 16, 16)] = _i32(_mixv(_u32(v)))
        loop(VPT, l1)
        pltpu.sync_copy(key_v, h_s.at[pl.ds(t * PT, PT)])
        plsc.subcore_barrier()
        pltpu.sync_copy(h_s, hfull_v)

        loop(NPAD // 16, lambda k: aggp_v.__setitem__(pl.ds(k * 16, 16), zeros))

        def l2(i):
            s = srcv[pl.ds(i * 16, 16)]
            d = dstv[pl.ds(i * 16, 16)]
            nb = plsc.load_gather(hfull_v, [s])
            plsc.addupdate_scatter(aggp_v, [d], nb)
        lax.fori_loop(0, ET // 16, lambda i, c: (l2(i), 0)[1], 0, unroll=4)

        pltpu.sync_copy(aggp_v, part_hbm.at[pl.ds(t * NPAD, NPAD)])
        plsc.subcore_barrier()
        cps = [pltpu.async_copy(
            part_hbm.at[pl.ds(p * NPAD + t * PT, PT)],
            aggp_v.at[pl.ds(p * PT, PT)], psem) for p in range(NT)]
        for cp in cps:
            cp.wait()

        def l3(k):
            acc = aggp_v[pl.ds(k * 16, 16)]
            for p in range(1, NT):
                acc = acc + aggp_v[pl.ds(p * PT + k * 16, 16)]
            dsti_v[pl.ds(k * 16, 16)] = acc
        loop(VPT, l3)


        def l4(k):
            a = dsti_v[pl.ds(k * 16, 16)]
            h = hfull_v[pl.ds(t * PT + k * 16, 16)]
            sig = _u32(h) * jnp.uint32(K1) + _mixv(_u32(a))
            gid = t * PT + k * 16 + lane
            sig = jnp.where(gid < N, sig, jnp.uint32(0xFFFFFFFF))
            key_v[pl.ds(k * 16, 16)] = _i32(sig)
            val_v[pl.ds(k * 16, 16)] = gid
        loop(VPT, l4)
        pltpu.sync_copy(key_v, keyA_s.at[pl.ds(t * PT, PT)])
        pltpu.sync_copy(val_v, valA_s.at[pl.ds(t * PT, PT)])

        for p in range(4):
            inK = keyA_s if p % 2 == 0 else keyB_s
            inV = valA_s if p % 2 == 0 else valB_s
            outK = keyB_s if p % 2 == 0 else keyA_s
            outV = valB_s if p % 2 == 0 else valA_s
            sh = 8 * p
            pltpu.sync_copy(inK.at[pl.ds(t * PT, PT)], key_v)
            pltpu.sync_copy(inV.at[pl.ds(t * PT, PT)], val_v)
            hist_v[pl.ds(0, 16)] = zeros
            loop(256 // 16,
                 lambda k: hist_v.__setitem__(pl.ds(k * 16, 16), zeros))

            def s1(k, _sh=sh):
                kv = key_v[pl.ds(k * 16, 16)]
                d = _i32((_u32(kv) >> _sh) & jnp.uint32(255))
                dig, _, rk, isend = runs(d * 16 + lane)
                plsc.addupdate_scatter(hist_v, [dig], rk + 1, mask=isend)
            loop(VPT, s1)
            pltpu.sync_copy(hist_v, ghist_s.at[pl.ds(t * 256, 256)])
            plsc.subcore_barrier()
            pltpu.sync_copy(ghist_s, grid_v)
            rcar = zeros
            for c in range(16):
                def s2(tp, a, _c=c):
                    accp, acct = a
                    v = grid_v[pl.ds(tp * 256 + _c * 16, 16)]
                    return (accp + jnp.where(tp < t, v, zeros), acct + v)
                accp, acct = lax.fori_loop(0, NT, s2, (zeros, zeros),
                                           unroll=4)
                cs = plsc.cumsum(acct)
                offs_v[pl.ds(c * 16, 16)] = accp + cs - acct + rcar
                rcar = rcar + _take(cs, l15)

            def s3(k, _sh=sh):
                kv = key_v[pl.ds(k * 16, 16)]
                d = _i32((_u32(kv) >> _sh) & jnp.uint32(255))
                dig, sl, rk, isend = runs(d * 16 + lane)
                base = plsc.load_gather(offs_v, [dig])
                plsc.addupdate_scatter(offs_v, [dig], rk + 1, mask=isend)
                plsc.store_scatter(dsti_v, [k * 16 + sl], base + rk)
            loop(VPT, s3)
            pltpu.sync_copy(key_v, outK.at[dsti_v])
            pltpu.sync_copy(val_v, outV.at[dsti_v])
            plsc.subcore_barrier()

        pltpu.sync_copy(keyA_s.at[pl.ds(t * PT, PT)], key_v)
        pltpu.sync_copy(valA_s.at[pl.ds(t * PT, PT)], val_v)
        off0 = pl.multiple_of(jnp.maximum(t * PT - 16, 0), 16)
        pltpu.sync_copy(keyA_s.at[pl.ds(off0, 16)], tmp16_v)
        kv0 = key_v[pl.ds(0, 16)]
        tl = tmp16_v[...]
        prev0 = jnp.where(t > 0, tl[15], kv0[0])

        def l6(k, c):
            runs_, prevs = c
            kv = key_v[pl.ds(k * 16, 16)]
            prevv = jnp.where(lane == 0, prevs, _take(kv, lanem1))
            nf = jnp.where(kv != prevv, ones, zeros)
            runv = runs_ + plsc.cumsum(nf)
            dsti_v[pl.ds(k * 16, 16)] = runv
            return _take(runv, l15), _take(kv, l15)
        cnt, _ = lax.fori_loop(0, VPT, l6,
                               (zeros, zeros + prev0))
        tmp16_v[...] = cnt
        pltpu.sync_copy(tmp16_v.at[pl.ds(0, 8)], tcount_s.at[pl.ds(t * 8, 8)])
        plsc.subcore_barrier()
        pltpu.sync_copy(tcount_s, hist_v.at[pl.ds(0, 128)])
        tcv = plsc.load_gather(hist_v, [lane * 8])
        off = jnp.sum(jnp.where(lane < t, tcv, zeros))
        loop(VPT, lambda k: dsti_v.__setitem__(
            pl.ds(k * 16, 16), dsti_v[pl.ds(k * 16, 16)] + off))
        pltpu.sync_copy(dsti_v, colors_s.at[val_v])
        plsc.subcore_barrier()
        return carry

    lax.fori_loop(0, 3, layer, 0)

    for zc in zcps:
        zc.wait()
    plsc.subcore_barrier()

    pltpu.sync_copy(colors_s.at[pl.ds(t * PT, PT)], col_v)
    loop(VPT, lambda k: key_v.__setitem__(pl.ds(k * 16, 16), ones))

    def f2(k):
        c = col_v[pl.ds(k * 16, 16)]
        b = batch_v[pl.ds(k * 16, 16)]
        gid = t * PT + k * 16 + lane
        idx = jnp.where(gid < N, b * NPAD + c, G * NPAD)
        dsti_v[pl.ds(k * 16, 16)] = idx
    loop(VPT, f2)
    pltpu.sync_copy(key_v, table_s.at[dsti_v], add=True)
    plsc.subcore_barrier()

    pltpu.sync_copy(table_s.at[pl.ds(t * 4 * NPAD, 4 * NPAD)],
                    tbl_hbm.at[pl.ds(t * 4 * NPAD, 4 * NPAD)])


@jax.jit
def _run(x, src, dst, batch):
    mesh = plsc.VectorSubcoreMesh(
        core_axis_name="c", subcore_axis_name="s", num_cores=1)
    f = pl.kernel(
        _body,
        out_type=jax.ShapeDtypeStruct((G * NPAD,), jnp.int32),
        mesh=mesh,
        compiler_params=pltpu.CompilerParams(needs_layout_passes=False),
        scratch_types=[
            pltpu.VMEM((NPAD,), jnp.int32),
            pltpu.VMEM((NPAD,), jnp.int32),
            pltpu.VMEM((NPAD,), jnp.int32),
            pltpu.VMEM((ET,), jnp.int32),
            pltpu.VMEM((ET,), jnp.int32),
            pltpu.VMEM((PT,), jnp.int32),
            pltpu.VMEM((PT,), jnp.int32),
            pltpu.VMEM((PT,), jnp.int32),
            pltpu.VMEM((PT,), jnp.int32),
            pltpu.VMEM((PT,), jnp.int32),
            pltpu.VMEM((PT,), jnp.int32),
            pltpu.VMEM((4096,), jnp.int32),
            pltpu.VMEM((256,), jnp.int32),
            pltpu.VMEM((256,), jnp.int32),
            pltpu.VMEM((16,), jnp.int32),
            pltpu.SemaphoreType.DMA,
            pltpu.SemaphoreType.DMA,
            pltpu.HBM((NT * NPAD,), jnp.int32),
            pltpu.VMEM_SHARED((NPAD,), jnp.int32),
            pltpu.VMEM_SHARED((NPAD,), jnp.int32),
            pltpu.VMEM_SHARED((NPAD,), jnp.int32),
            pltpu.VMEM_SHARED((NPAD,), jnp.int32),
            pltpu.VMEM_SHARED((NPAD,), jnp.int32),
            pltpu.VMEM_SHARED((NPAD,), jnp.int32),
            pltpu.VMEM_SHARED((4096,), jnp.int32),
            pltpu.VMEM_SHARED((128,), jnp.int32),
            pltpu.VMEM_SHARED((G * NPAD + 8,), jnp.int32),
        ],
    )
    return f(x, src, dst, batch)


def _tc_body(hist_ref, w_ref, o_ref):
    h = hist_ref[...].astype(jnp.float32)
    nrm2 = jnp.sum(h * h, axis=1, keepdims=True)
    out = jnp.dot(h[:, :N], w_ref[...], preferred_element_type=jnp.float32,
                  precision=lax.Precision.HIGHEST)
    nrm = jnp.sqrt(nrm2)
    o_ref[...] = out / jnp.where(nrm > 0.0, nrm, 1.0)


@jax.jit
def _project(hist, w_pad):
    return pl.pallas_call(
        _tc_body,
        out_shape=jax.ShapeDtypeStruct((G, C_OUT), jnp.float32),
    )(hist, w_pad)


def kernel(x, edge_index, batch, W, b):
    src = edge_index[0].astype(jnp.int32)
    dst = edge_index[1].astype(jnp.int32)
    tbl = _run(x.astype(jnp.int32), src, dst, batch.astype(jnp.int32))
    return _project(tbl.reshape(G, NPAD), W) + b


# speedup vs baseline: 1.0173x; 1.0091x over previous
import jax
import jax.numpy as jnp
from jax import lax
from jax.experimental import pallas as pl
from jax.experimental.pallas import tpu as pltpu
from jax.experimental.pallas import tpu_sc as plsc

N = 10000
NPAD = 10240
E = 320000
G = 64
C_OUT = 128
NT = 16
PT = NPAD // NT
ET = E // NT
VPT = PT // 16
WCH = 160
K1 = 0x9E3779B9


def _mixv(u):
    u = (u ^ (u >> 16)) * jnp.uint32(0x7FEB352D)
    u = (u ^ (u >> 15)) * jnp.uint32(0x846CA68B)
    return u ^ (u >> 16)


def _u32(x):
    return plsc.bitcast(x, jnp.uint32)


def _i32(x):
    return plsc.bitcast(x, jnp.int32)


_DNUMS = lax.GatherDimensionNumbers(
    offset_dims=(), collapsed_slice_dims=(0,), start_index_map=(0,))


def _take(v, idx):
    return lax.gather(v, idx[:, None], dimension_numbers=_DNUMS,
                      slice_sizes=(1,),
                      mode=lax.GatherScatterMode.PROMISE_IN_BOUNDS)


def _body(x_hbm, src_hbm, dst_hbm, batch_hbm,
          tbl_hbm,
          zsrc_v, hfull_v, aggp_v, srcv, dstv,
          slice_v, key_v, val_v, dsti_v, col_v, batch_v,
          grid_v, hist_v, offs_v, tmp16_v, zsem, psem,
          part_hbm,
          colors_s, h_s, keyA_s, valA_s, keyB_s, valB_s,
          ghist_s, tcount_s, table_s):
    t = lax.axis_index("s")
    lane = lax.iota(jnp.int32, 16)
    ones = jnp.ones((16,), jnp.int32)
    zeros = jnp.zeros((16,), jnp.int32)
    lanem1 = jnp.maximum(lane - 1, 0)
    lanep1 = jnp.minimum(lane + 1, 15)
    l15 = jnp.full((16,), 15, jnp.int32)

    def loop(n, body):
        lax.fori_loop(0, n, lambda i, c: (body(i), 0)[1], 0)

    def runs(sortkey):
        srt = lax.sort(sortkey)
        dig = srt >> 4
        sl = srt & 15
        prev = _take(dig, lanem1)
        isstart = jnp.logical_or(lane == 0, dig != prev)
        st = plsc.cummax(jnp.where(isstart, lane, zeros))
        nxt = _take(dig, lanep1)
        isend = jnp.logical_or(lane == 15, dig != nxt)
        return dig, sl, lane - st, isend

    LAST = N - 15 * PT

    @pl.when(t < 15)
    def _cf():
        pltpu.sync_copy(x_hbm.at[pl.ds(t * PT, PT)], slice_v)
        pltpu.sync_copy(batch_hbm.at[pl.ds(t * PT, PT)], batch_v)

    @pl.when(t == 15)
    def _cl():
        pltpu.sync_copy(x_hbm.at[pl.ds(15 * PT, LAST)],
                        slice_v.at[pl.ds(0, LAST)])
        pltpu.sync_copy(batch_hbm.at[pl.ds(15 * PT, LAST)],
                        batch_v.at[pl.ds(0, LAST)])
    pltpu.sync_copy(slice_v, colors_s.at[pl.ds(t * PT, PT)])
    pltpu.sync_copy(src_hbm.at[pl.ds(t * ET, ET)], srcv)
    pltpu.sync_copy(dst_hbm.at[pl.ds(t * ET, ET)], dstv)
    loop(NPAD // 16, lambda k: zsrc_v.__setitem__(pl.ds(k * 16, 16), zeros))
    zcps = [pltpu.async_copy(
        zsrc_v, table_s.at[pl.ds(t * 4 * NPAD + j * NPAD, NPAD)], zsem)
        for j in range(4)]

    @pl.when(t == 0)
    def _zt0():
        pltpu.sync_copy(zsrc_v.at[pl.ds(0, 8)], table_s.at[pl.ds(G * NPAD, 8)])
    plsc.subcore_barrier()

    def layer(_l, carry):
        pltpu.sync_copy(colors_s.at[pl.ds(t * PT, PT)], slice_v)

        def l1(k):
            v = slice_v[pl.ds(k * 16, 16)]
            key_v[pl.ds(k * 16, 16)] = _i32(_mixv(_u32(v)))
        loop(VPT, l1)
        pltpu.sync_copy(key_v, h_s.at[pl.ds(t * PT, PT)])
        plsc.subcore_barrier()
        pltpu.sync_copy(h_s, hfull_v)

        loop(NPAD // 16, lambda k: aggp_v.__setitem__(pl.ds(k * 16, 16), zeros))

        def l2(i):
            s = srcv[pl.ds(i * 16, 16)]
            d = dstv[pl.ds(i * 16, 16)]
            nb = plsc.load_gather(hfull_v, [s])
            plsc.addupdate_scatter(aggp_v, [d], nb)
        lax.fori_loop(0, ET // 16, lambda i, c: (l2(i), 0)[1], 0, unroll=4)

        pltpu.sync_copy(aggp_v, part_hbm.at[pl.ds(t * NPAD, NPAD)])
        plsc.subcore_barrier()
        cps = [pltpu.async_copy(
            part_hbm.at[pl.ds(p * NPAD + t * PT, PT)],
            aggp_v.at[pl.ds(p * PT, PT)], psem) for p in range(NT)]
        for cp in cps:
            cp.wait()

        def l3(k):
            acc = aggp_v[pl.ds(k * 16, 16)]
            for p in range(1, NT):
                acc = acc + aggp_v[pl.ds(p * PT + k * 16, 16)]
            dsti_v[pl.ds(k * 16, 16)] = acc
        loop(VPT, l3)


        def l4(k):
            a = dsti_v[pl.ds(k * 16, 16)]
            h = hfull_v[pl.ds(t * PT + k * 16, 16)]
            sig = _u32(h) * jnp.uint32(K1) + _mixv(_u32(a))
            gid = t * PT + k * 16 + lane
            sig = jnp.where(gid < N, sig, jnp.uint32(0xFFFFFFFF))
            key_v[pl.ds(k * 16, 16)] = _i32(sig)
            val_v[pl.ds(k * 16, 16)] = gid
        loop(VPT, l4)
        pltpu.sync_copy(key_v, keyA_s.at[pl.ds(t * PT, PT)])
        pltpu.sync_copy(val_v, valA_s.at[pl.ds(t * PT, PT)])

        for p in range(4):
            inK = keyA_s if p % 2 == 0 else keyB_s
            inV = valA_s if p % 2 == 0 else valB_s
            outK = keyB_s if p % 2 == 0 else keyA_s
            outV = valB_s if p % 2 == 0 else valA_s
            sh = 8 * p
            pltpu.sync_copy(inK.at[pl.ds(t * PT, PT)], key_v)
            pltpu.sync_copy(inV.at[pl.ds(t * PT, PT)], val_v)
            hist_v[pl.ds(0, 16)] = zeros
            loop(256 // 16,
                 lambda k: hist_v.__setitem__(pl.ds(k * 16, 16), zeros))

            def s1(k, _sh=sh):
                kv = key_v[pl.ds(k * 16, 16)]
                d = _i32((_u32(kv) >> _sh) & jnp.uint32(255))
                dig, _, rk, isend = runs(d * 16 + lane)
                plsc.addupdate_scatter(hist_v, [dig], rk + 1, mask=isend)
            loop(VPT, s1)
            pltpu.sync_copy(hist_v, ghist_s.at[pl.ds(t * 256, 256)])
            plsc.subcore_barrier()
            pltpu.sync_copy(ghist_s, grid_v)
            rcar = zeros
            for c in range(16):
                def s2(tp, a, _c=c):
                    accp, acct = a
                    v = grid_v[pl.ds(tp * 256 + _c * 16, 16)]
                    return (accp + jnp.where(tp < t, v, zeros), acct + v)
                accp, acct = lax.fori_loop(0, NT, s2, (zeros, zeros),
                                           unroll=4)
                cs = plsc.cumsum(acct)
                offs_v[pl.ds(c * 16, 16)] = accp + cs - acct + rcar
                rcar = rcar + _take(cs, l15)

            def s3(k, _sh=sh):
                kv = key_v[pl.ds(k * 16, 16)]
                d = _i32((_u32(kv) >> _sh) & jnp.uint32(255))
                dig, sl, rk, isend = runs(d * 16 + lane)
                base = plsc.load_gather(offs_v, [dig])
                plsc.addupdate_scatter(offs_v, [dig], rk + 1, mask=isend)
                plsc.store_scatter(dsti_v, [k * 16 + sl], base + rk)
            loop(VPT, s3)
            pltpu.sync_copy(key_v, outK.at[dsti_v])
            pltpu.sync_copy(val_v, outV.at[dsti_v])
            plsc.subcore_barrier()

        pltpu.sync_copy(keyA_s.at[pl.ds(t * PT, PT)], key_v)
        pltpu.sync_copy(valA_s.at[pl.ds(t * PT, PT)], val_v)
        off0 = pl.multiple_of(jnp.maximum(t * PT - 16, 0), 16)
        pltpu.sync_copy(keyA_s.at[pl.ds(off0, 16)], tmp16_v)
        kv0 = key_v[pl.ds(0, 16)]
        tl = tmp16_v[...]
        prev0 = jnp.where(t > 0, tl[15], kv0[0])

        def l6(k, c):
            runs_, prevs = c
            kv = key_v[pl.ds(k * 16, 16)]
            prevv = jnp.where(lane == 0, prevs, _take(kv, lanem1))
            nf = jnp.where(kv != prevv, ones, zeros)
            runv = runs_ + plsc.cumsum(nf)
            dsti_v[pl.ds(k * 16, 16)] = runv
            return _take(runv, l15), _take(kv, l15)
        cnt, _ = lax.fori_loop(0, VPT, l6,
                               (zeros, zeros + prev0))
        tmp16_v[...] = cnt
        pltpu.sync_copy(tmp16_v.at[pl.ds(0, 8)], tcount_s.at[pl.ds(t * 8, 8)])
        plsc.subcore_barrier()
        pltpu.sync_copy(tcount_s, hist_v.at[pl.ds(0, 128)])
        tcv = plsc.load_gather(hist_v, [lane * 8])
        off = jnp.sum(jnp.where(lane < t, tcv, zeros))
        loop(VPT, lambda k: dsti_v.__setitem__(
            pl.ds(k * 16, 16), dsti_v[pl.ds(k * 16, 16)] + off))
        pltpu.sync_copy(dsti_v, colors_s.at[val_v])
        plsc.subcore_barrier()
        return carry

    lax.fori_loop(0, 3, layer, 0)

    for zc in zcps:
        zc.wait()
    plsc.subcore_barrier()

    pltpu.sync_copy(colors_s.at[pl.ds(t * PT, PT)], col_v)
    loop(VPT, lambda k: key_v.__setitem__(pl.ds(k * 16, 16), ones))

    def f2(k):
        c = col_v[pl.ds(k * 16, 16)]
        b = batch_v[pl.ds(k * 16, 16)]
        gid = t * PT + k * 16 + lane
        idx = jnp.where(gid < N, b * NPAD + c, G * NPAD)
        dsti_v[pl.ds(k * 16, 16)] = idx
    loop(VPT, f2)
    pltpu.sync_copy(key_v, table_s.at[dsti_v], add=True)
    plsc.subcore_barrier()

    pltpu.sync_copy(table_s.at[pl.ds(t * 4 * NPAD, 4 * NPAD)],
                    tbl_hbm.at[pl.ds(t * 4 * NPAD, 4 * NPAD)])


@jax.jit
def _run(x, src, dst, batch):
    mesh = plsc.VectorSubcoreMesh(
        core_axis_name="c", subcore_axis_name="s", num_cores=1)
    f = pl.kernel(
        _body,
        out_type=jax.ShapeDtypeStruct((G * NPAD,), jnp.int32),
        mesh=mesh,
        compiler_params=pltpu.CompilerParams(needs_layout_passes=False),
        scratch_types=[
            pltpu.VMEM((NPAD,), jnp.int32),
            pltpu.VMEM((NPAD,), jnp.int32),
            pltpu.VMEM((NPAD,), jnp.int32),
            pltpu.VMEM((ET,), jnp.int32),
            pltpu.VMEM((ET,), jnp.int32),
            pltpu.VMEM((PT,), jnp.int32),
            pltpu.VMEM((PT,), jnp.int32),
            pltpu.VMEM((PT,), jnp.int32),
            pltpu.VMEM((PT,), jnp.int32),
            pltpu.VMEM((PT,), jnp.int32),
            pltpu.VMEM((PT,), jnp.int32),
            pltpu.VMEM((4096,), jnp.int32),
            pltpu.VMEM((256,), jnp.int32),
            pltpu.VMEM((256,), jnp.int32),
            pltpu.VMEM((16,), jnp.int32),
            pltpu.SemaphoreType.DMA,
            pltpu.SemaphoreType.DMA,
            pltpu.HBM((NT * NPAD,), jnp.int32),
            pltpu.VMEM_SHARED((NPAD,), jnp.int32),
            pltpu.VMEM_SHARED((NPAD,), jnp.int32),
            pltpu.VMEM_SHARED((NPAD,), jnp.int32),
            pltpu.VMEM_SHARED((NPAD,), jnp.int32),
            pltpu.VMEM_SHARED((NPAD,), jnp.int32),
            pltpu.VMEM_SHARED((NPAD,), jnp.int32),
            pltpu.VMEM_SHARED((4096,), jnp.int32),
            pltpu.VMEM_SHARED((128,), jnp.int32),
            pltpu.VMEM_SHARED((G * NPAD + 8,), jnp.int32),
        ],
    )
    return f(x, src, dst, batch)


def _tc_body(hist_ref, w_ref, b_ref, o_ref):
    h = hist_ref[...].astype(jnp.float32)
    nrm2 = jnp.sum(h * h, axis=1, keepdims=True)
    out = jnp.dot(h[:, :N], w_ref[...], preferred_element_type=jnp.float32,
                  precision=lax.Precision.HIGHEST)
    nrm = jnp.sqrt(nrm2)
    o_ref[...] = out / jnp.where(nrm > 0.0, nrm, 1.0) + b_ref[...]


@jax.jit
def _project(hist, w, b):
    return pl.pallas_call(
        _tc_body,
        out_shape=jax.ShapeDtypeStruct((G, C_OUT), jnp.float32),
    )(hist, w, b.reshape(1, C_OUT))


def kernel(x, edge_index, batch, W, b):
    tbl = _run(x.astype(jnp.int32), edge_index[0], edge_index[1],
               batch.astype(jnp.int32))
    return _project(tbl.reshape(G, NPAD), W, b)
